# baseline (speedup 1.0000x reference)
import functools

import jax
import jax.numpy as jnp
from jax import lax
from jax.experimental import pallas as pl
from jax.experimental.pallas import tpu as pltpu
from jax.experimental.pallas import tpu_sc as plsc

B, T, V = 32, 16, 100000
N = B * T
NS = 16
ROWS = N // NS
L = 16
CHUNKS = ROWS // L
SEG = 128


_mesh = plsc.VectorSubcoreMesh(
    core_axis_name="c", subcore_axis_name="s", num_cores=1)

_SCRATCH = [
    pltpu.VMEM((B, T), jnp.int32),
    pltpu.VMEM((B, T), jnp.float32),
    pltpu.VMEM((ROWS, 8, SEG), jnp.float32),
    pltpu.VMEM((ROWS, 8, SEG), jnp.float32),
    pltpu.VMEM((2, L), jnp.float32),
    pltpu.VMEM((L,), jnp.float32),
    pltpu.VMEM((NS, 2, L), jnp.float32),
    pltpu.HBM((NS, 2, L), jnp.float32),
    pltpu.SemaphoreType.DMA,
    pltpu.SemaphoreType.DMA,
]


def _nll_body(fine_hbm, final_hbm, tgt_hbm, msk_hbm, out_hbm,
              tgt_all, msk_all, fine_seg, final_seg, stage_v, res_v, red_v,
              bounce_hbm, sem_a, sem_b):
    sid = lax.axis_index("s")

    pltpu.sync_copy(tgt_hbm, tgt_all)
    pltpu.sync_copy(msk_hbm, msk_all)

    lane = lax.iota(jnp.int32, L)
    subl = lane & 7
    lacc = jnp.zeros((L,), jnp.float32)
    macc = jnp.zeros((L,), jnp.float32)
    tchunks = []
    descs = []
    for k in range(CHUNKS):
        b = sid * CHUNKS + k
        tchunk = tgt_all[b]
        tchunks.append(tchunk)
        for j in range(L):
            t0 = j & ~7
            c0 = pl.multiple_of((tchunk[j] >> 7) << 7, SEG)
            r = k * L + j
            d1 = pltpu.make_async_copy(
                fine_hbm.at[b, pl.ds(t0, 8), pl.ds(c0, SEG)],
                fine_seg.at[r], sem_a)
            d2 = pltpu.make_async_copy(
                final_hbm.at[b, pl.ds(t0, 8), pl.ds(c0, SEG)],
                final_seg.at[r], sem_b)
            d1.start()
            d2.start()
            descs.append(d1)
            descs.append(d2)
    for d in descs:
        d.wait()

    for k in range(CHUNKS):
        b = sid * CHUNKS + k
        tchunk = tchunks[k]
        colv = tchunk & 127
        rowv = lane + (k * L)
        fvals = plsc.load_gather(fine_seg, [rowv, subl, colv])
        gvals = plsc.load_gather(final_seg, [rowv, subl, colv])
        m = msk_all[b]
        lacc = lacc + (fvals + gvals) * m
        macc = macc + m

    stage_v[0] = lacc
    stage_v[1] = macc
    pltpu.sync_copy(stage_v, bounce_hbm.at[sid])
    plsc.subcore_barrier()

    @pl.when(sid == 0)
    def _finish():
        pltpu.sync_copy(bounce_hbm, red_v)
        lsum = jnp.zeros((L,), jnp.float32)
        msum = jnp.zeros((L,), jnp.float32)
        for r in range(NS):
            lsum = lsum + red_v[r, 0]
            msum = msum + red_v[r, 1]
        ltot = jnp.float32(0.0)
        mtot = jnp.float32(0.0)
        for i in range(L):
            ltot = ltot + lsum[i]
            mtot = mtot + msum[i]
        res_v[...] = jnp.broadcast_to(-ltot, (L,)) / jnp.broadcast_to(mtot, (L,))
        pltpu.sync_copy(res_v, out_hbm)


_nll_kernel = functools.partial(
    pl.kernel,
    out_type=jax.ShapeDtypeStruct((L,), jnp.float32),
    mesh=_mesh,
    scratch_types=_SCRATCH,
    compiler_params=pltpu.CompilerParams(needs_layout_passes=False),
)(_nll_body)


def kernel(input_fine, input_final, target, mask):
    out = _nll_kernel(input_fine, input_final, target, mask)
    return out[0]


# speedup vs baseline: 1.0027x; 1.0027x over previous
import functools

import jax
import jax.numpy as jnp
from jax import lax
from jax.experimental import pallas as pl
from jax.experimental.pallas import tpu as pltpu
from jax.experimental.pallas import tpu_sc as plsc

B, T, V = 32, 16, 100000
N = B * T
NS = 16
ROWS = N // NS
L = 16
CHUNKS = ROWS // L
SEG = 128


_mesh = plsc.VectorSubcoreMesh(
    core_axis_name="c", subcore_axis_name="s", num_cores=1)

_SCRATCH = [
    pltpu.VMEM((T, B), jnp.int32),
    pltpu.VMEM((T, B), jnp.float32),
    pltpu.VMEM((ROWS, 8, SEG), jnp.float32),
    pltpu.VMEM((ROWS, 8, SEG), jnp.float32),
    pltpu.VMEM((2, L), jnp.float32),
    pltpu.VMEM((L,), jnp.float32),
    pltpu.VMEM((NS, 2, L), jnp.float32),
    pltpu.HBM((NS, 2, L), jnp.float32),
    pltpu.SemaphoreType.DMA,
    pltpu.SemaphoreType.DMA,
]


def _nll_body(fine_hbm, final_hbm, tgt_hbm, msk_hbm, out_hbm,
              tgt_all, msk_all, fine_seg, final_seg, stage_v, res_v, red_v,
              bounce_hbm, sem_a, sem_b):
    sid = lax.axis_index("s")

    pltpu.sync_copy(tgt_hbm, tgt_all)
    pltpu.sync_copy(msk_hbm, msk_all)

    lane = lax.iota(jnp.int32, L)
    subl = lane & 7
    lacc = jnp.zeros((L,), jnp.float32)
    macc = jnp.zeros((L,), jnp.float32)
    tchunks = []
    descs = []
    for k in range(CHUNKS):
        b = sid * CHUNKS + k
        bvec = jnp.broadcast_to(b, (L,)).astype(jnp.int32)
        tchunk = plsc.load_gather(tgt_all, [lane, bvec])
        tchunks.append(tchunk)
        for j in range(L):
            t0 = j & ~7
            c0 = pl.multiple_of((tchunk[j] >> 7) << 7, SEG)
            r = k * L + j
            d1 = pltpu.make_async_copy(
                fine_hbm.at[b, pl.ds(t0, 8), pl.ds(c0, SEG)],
                fine_seg.at[r], sem_a)
            d2 = pltpu.make_async_copy(
                final_hbm.at[b, pl.ds(t0, 8), pl.ds(c0, SEG)],
                final_seg.at[r], sem_b)
            d1.start()
            d2.start()
            descs.append(d1)
            descs.append(d2)
    for d in descs:
        d.wait()

    for k in range(CHUNKS):
        b = sid * CHUNKS + k
        bvec = jnp.broadcast_to(b, (L,)).astype(jnp.int32)
        tchunk = tchunks[k]
        colv = tchunk & 127
        rowv = lane + (k * L)
        fvals = plsc.load_gather(fine_seg, [rowv, subl, colv])
        gvals = plsc.load_gather(final_seg, [rowv, subl, colv])
        m = plsc.load_gather(msk_all, [lane, bvec])
        lacc = lacc + (fvals + gvals) * m
        macc = macc + m

    stage_v[0] = lacc
    stage_v[1] = macc
    pltpu.sync_copy(stage_v, bounce_hbm.at[sid])
    plsc.subcore_barrier()

    @pl.when(sid == 0)
    def _finish():
        pltpu.sync_copy(bounce_hbm, red_v)
        lsum = jnp.zeros((L,), jnp.float32)
        msum = jnp.zeros((L,), jnp.float32)
        for r in range(NS):
            lsum = lsum + red_v[r, 0]
            msum = msum + red_v[r, 1]
        ltot = jnp.float32(0.0)
        mtot = jnp.float32(0.0)
        for i in range(L):
            ltot = ltot + lsum[i]
            mtot = mtot + msum[i]
        res_v[...] = jnp.broadcast_to(-ltot, (L,)) / jnp.broadcast_to(mtot, (L,))
        pltpu.sync_copy(res_v, out_hbm)


_nll_kernel = functools.partial(
    pl.kernel,
    out_type=jax.ShapeDtypeStruct((L,), jnp.float32),
    mesh=_mesh,
    scratch_types=_SCRATCH,
    compiler_params=pltpu.CompilerParams(needs_layout_passes=False),
)(_nll_body)


def kernel(input_fine, input_final, target, mask):
    out = _nll_kernel(input_fine, input_final, target.T, mask.T)
    return out[0]


# speedup vs baseline: 1.0491x; 1.0462x over previous
import functools

import jax
import jax.numpy as jnp
from jax import lax
from jax.experimental import pallas as pl
from jax.experimental.pallas import tpu as pltpu
from jax.experimental.pallas import tpu_sc as plsc

B, T, V = 32, 16, 100000
N = B * T
NS = 16
ROWS = N // NS
L = 16
CHUNKS = ROWS // L
SEG = 128


_mesh = plsc.VectorSubcoreMesh(
    core_axis_name="c", subcore_axis_name="s", num_cores=1)

_SCRATCH = [
    pltpu.VMEM((T, B), jnp.int32),
    pltpu.VMEM((T, B), jnp.float32),
    pltpu.VMEM((ROWS, 8, SEG), jnp.float32),
    pltpu.VMEM((ROWS, 8, SEG), jnp.float32),
    pltpu.VMEM((2, L), jnp.float32),
    pltpu.VMEM((L,), jnp.float32),
    pltpu.VMEM((NS, 2, L), jnp.float32),
    pltpu.HBM((NS, 2, L), jnp.float32),
    pltpu.SemaphoreType.DMA,
    pltpu.SemaphoreType.DMA,
    pltpu.SemaphoreType.DMA,
]


def _nll_body(fine_hbm, final_hbm, tgt_hbm, msk_hbm, out_hbm,
              tgt_all, msk_all, fine_seg, final_seg, stage_v, res_v, red_v,
              bounce_hbm, sem_a, sem_b, sem_c):
    sid = lax.axis_index("s")

    pltpu.sync_copy(tgt_hbm, tgt_all)
    msk_cp = pltpu.make_async_copy(msk_hbm, msk_all, sem_c)
    msk_cp.start()

    lane = lax.iota(jnp.int32, L)
    subl = lane & 7
    lacc = jnp.zeros((L,), jnp.float32)
    macc = jnp.zeros((L,), jnp.float32)
    tchunks = []
    descs = []
    for k in range(CHUNKS):
        b = sid * CHUNKS + k
        bvec = jnp.broadcast_to(b, (L,)).astype(jnp.int32)
        tchunk = plsc.load_gather(tgt_all, [lane, bvec])
        tchunks.append(tchunk)
        for j in range(L):
            t0 = j & ~7
            c0 = pl.multiple_of((tchunk[j] >> 7) << 7, SEG)
            r = k * L + j
            d1 = pltpu.make_async_copy(
                fine_hbm.at[b, pl.ds(t0, 8), pl.ds(c0, SEG)],
                fine_seg.at[r], sem_a)
            d2 = pltpu.make_async_copy(
                final_hbm.at[b, pl.ds(t0, 8), pl.ds(c0, SEG)],
                final_seg.at[r], sem_b)
            d1.start()
            d2.start()
            descs.append(d1)
            descs.append(d2)
    msk_cp.wait()
    for d in descs:
        d.wait()

    for k in range(CHUNKS):
        b = sid * CHUNKS + k
        bvec = jnp.broadcast_to(b, (L,)).astype(jnp.int32)
        tchunk = tchunks[k]
        colv = tchunk & 127
        rowv = lane + (k * L)
        fvals = plsc.load_gather(fine_seg, [rowv, subl, colv])
        gvals = plsc.load_gather(final_seg, [rowv, subl, colv])
        m = plsc.load_gather(msk_all, [lane, bvec])
        lacc = lacc + (fvals + gvals) * m
        macc = macc + m

    stage_v[0] = lacc
    stage_v[1] = macc
    pltpu.sync_copy(stage_v, bounce_hbm.at[sid])
    plsc.subcore_barrier()

    @pl.when(sid == 0)
    def _finish():
        pltpu.sync_copy(bounce_hbm, red_v)
        lsum = jnp.zeros((L,), jnp.float32)
        msum = jnp.zeros((L,), jnp.float32)
        for r in range(NS):
            lsum = lsum + red_v[r, 0]
            msum = msum + red_v[r, 1]
        ltot = plsc.cumsum(lsum)[L - 1]
        mtot = plsc.cumsum(msum)[L - 1]
        res_v[...] = jnp.broadcast_to(-ltot, (L,)) / jnp.broadcast_to(mtot, (L,))
        pltpu.sync_copy(res_v, out_hbm)


_nll_kernel = functools.partial(
    pl.kernel,
    out_type=jax.ShapeDtypeStruct((L,), jnp.float32),
    mesh=_mesh,
    scratch_types=_SCRATCH,
    compiler_params=pltpu.CompilerParams(needs_layout_passes=False),
)(_nll_body)


def kernel(input_fine, input_final, target, mask):
    out = _nll_kernel(input_fine, input_final, target.T, mask.T)
    return out[0]


# speedup vs baseline: 1.0550x; 1.0057x over previous
import functools

import jax
import jax.numpy as jnp
from jax import lax
from jax.experimental import pallas as pl
from jax.experimental.pallas import tpu as pltpu
from jax.experimental.pallas import tpu_sc as plsc

B, T, V = 32, 16, 100000
N = B * T
NS = 16
ROWS = N // NS
L = 16
CHUNKS = ROWS // L
SEG = 128


_SCRATCH = [
    pltpu.VMEM((T, B), jnp.int32),
    pltpu.VMEM((T, B), jnp.float32),
    pltpu.VMEM((ROWS, 8, SEG), jnp.float32),
    pltpu.VMEM((ROWS, 8, SEG), jnp.float32),
    pltpu.VMEM((2, L), jnp.float32),
    pltpu.VMEM((L,), jnp.float32),
    pltpu.VMEM((NS, 2, L), jnp.float32),
    pltpu.HBM((NS, 2, L), jnp.float32),
    pltpu.SemaphoreType.DMA,
    pltpu.SemaphoreType.DMA,
    pltpu.SemaphoreType.DMA,
]


def _nll_body(fine_hbm, final_hbm, tgt_hbm, msk_hbm, out_hbm,
              tgt_all, msk_all, fine_seg, final_seg, stage_v, res_v, red_v,
              bounce_hbm, sem_a, sem_b, sem_c):
    sid = lax.axis_index("s")

    pltpu.sync_copy(tgt_hbm, tgt_all)
    msk_cp = pltpu.make_async_copy(msk_hbm, msk_all, sem_c)
    msk_cp.start()

    lane = lax.iota(jnp.int32, L)
    subl = lane & 7
    lacc = jnp.zeros((L,), jnp.float32)
    macc = jnp.zeros((L,), jnp.float32)
    tchunks = []
    descs = []
    for k in range(CHUNKS):
        b = sid * CHUNKS + k
        bvec = jnp.broadcast_to(b, (L,)).astype(jnp.int32)
        tchunk = plsc.load_gather(tgt_all, [lane, bvec])
        tchunks.append(tchunk)
        for j in range(L):
            t0 = j & ~7
            c0 = pl.multiple_of((tchunk[j] >> 7) << 7, SEG)
            r = k * L + j
            d1 = pltpu.make_async_copy(
                fine_hbm.at[b, pl.ds(t0, 8), pl.ds(c0, SEG)],
                fine_seg.at[r], sem_a)
            d2 = pltpu.make_async_copy(
                final_hbm.at[b, pl.ds(t0, 8), pl.ds(c0, SEG)],
                final_seg.at[r], sem_b)
            d1.start()
            d2.start()
            descs.append(d1)
            descs.append(d2)
    msk_cp.wait()
    for d in descs:
        d.wait()

    for k in range(CHUNKS):
        b = sid * CHUNKS + k
        bvec = jnp.broadcast_to(b, (L,)).astype(jnp.int32)
        tchunk = tchunks[k]
        colv = tchunk & 127
        rowv = lane + (k * L)
        fvals = plsc.load_gather(fine_seg, [rowv, subl, colv])
        gvals = plsc.load_gather(final_seg, [rowv, subl, colv])
        m = plsc.load_gather(msk_all, [lane, bvec])
        lacc = lacc + (fvals + gvals) * m
        macc = macc + m

    stage_v[0] = lacc
    stage_v[1] = macc
    pltpu.sync_copy(stage_v, bounce_hbm.at[sid])
    plsc.subcore_barrier()

    @pl.when(sid == 0)
    def _finish():
        pltpu.sync_copy(bounce_hbm, red_v)
        lsum = jnp.zeros((L,), jnp.float32)
        msum = jnp.zeros((L,), jnp.float32)
        for r in range(NS):
            lsum = lsum + red_v[r, 0]
            msum = msum + red_v[r, 1]
        ltot = plsc.cumsum(lsum)[L - 1]
        mtot = plsc.cumsum(msum)[L - 1]
        res_v[...] = jnp.broadcast_to(-ltot, (L,)) / jnp.broadcast_to(mtot, (L,))
        pltpu.sync_copy(res_v, out_hbm)


@functools.cache
def _nll_kernel():
    mesh = plsc.VectorSubcoreMesh(
        core_axis_name="c", subcore_axis_name="s", num_cores=1, num_subcores=NS)
    return functools.partial(
        pl.kernel,
        out_type=jax.ShapeDtypeStruct((L,), jnp.float32),
        mesh=mesh,
        scratch_types=_SCRATCH,
        compiler_params=pltpu.CompilerParams(needs_layout_passes=False),
    )(_nll_body)


def kernel(input_fine, input_final, target, mask):
    out = _nll_kernel()(input_fine, input_final, target.T, mask.T)
    return out[0]
